# baseline (speedup 1.0000x reference)
import jax
import jax.numpy as jnp
from jax.experimental import pallas as pl
from jax.experimental.pallas import tpu as pltpu
from jax.experimental.pallas import tpu_sc as plsc

D_MODEL = 128
SCALE = float(D_MODEL) ** 0.5
GATHER_WINDOW = 128
B_TC = 16


def _sc_gather(table, indices):
    num_indices = indices.shape[1]
    mesh = plsc.VectorSubcoreMesh(core_axis_name="c", subcore_axis_name="s")

    @pl.kernel(
        out_type=jax.ShapeDtypeStruct((num_indices, D_MODEL), table.dtype),
        mesh=mesh,
    )
    def k(table_hbm, idx_hbm, out_hbm):
        def body(idx_vmem, out_vmem):
            pltpu.sync_copy(table_hbm.at[idx_vmem.at[0]], out_vmem)

        pltpu.emit_pipeline(
            body,
            grid=(num_indices // GATHER_WINDOW,),
            in_specs=[
                pl.BlockSpec((1, GATHER_WINDOW), index_map=lambda i: (0, i))
            ],
            out_specs=[
                pl.BlockSpec((GATHER_WINDOW, D_MODEL), index_map=lambda i: (i, 0))
            ],
            core_axis_name=("c", "s"),
            dimension_semantics=(pltpu.PARALLEL,),
        )(idx_hbm, out_hbm)

    return k(table, indices)


def _scale_reshape(flat, batch, seq):
    def body(t_ref, o_ref):
        for b in range(B_TC):
            o_ref[b, :, :] = t_ref[pl.ds(b * seq, seq), :] * SCALE

    return pl.pallas_call(
        body,
        grid=(batch // B_TC,),
        in_specs=[pl.BlockSpec((B_TC * seq, D_MODEL), lambda i: (i, 0))],
        out_specs=pl.BlockSpec((B_TC, seq, D_MODEL), lambda i: (i, 0, 0)),
        out_shape=jax.ShapeDtypeStruct((batch, seq, D_MODEL), flat.dtype),
    )(flat)


def kernel(x, emb_weight):
    batch, seq = x.shape
    flat_idx = x.reshape(1, -1).astype(jnp.int32)
    flat = _sc_gather(emb_weight, flat_idx)
    return _scale_reshape(flat, batch, seq)
